# baseline (speedup 1.0000x reference)
import functools

import jax
import jax.numpy as jnp
from jax import lax
from jax.experimental import pallas as pl
from jax.experimental.pallas import tpu as pltpu
from jax.experimental.pallas import tpu_sc as plsc

_VOCAB = 100000
_SEQ = 200
_D = 64
_C = 10000
_BATCH = 4096

_NC = 2
_NS = 16
_NW = _NC * _NS

_BPW = _BATCH // _NW
_DT = _D // 8
_SC2 = 2
_NCHUNK = _SEQ // _SC2
_PAIRS = _NCHUNK // 2
_LANES = 16
_VPR = _D // _LANES


def _positional(seq_len, d_model, c):
    pos = jnp.arange(1, seq_len + 1, dtype=jnp.float32)[:, None]
    j = jnp.arange(d_model)[None, :]
    k = (j + 1) // 2
    angle = pos / jnp.power(jnp.float32(c), k.astype(jnp.float32) / d_model)
    return jnp.where((j % 2) == 0, jnp.sin(angle), jnp.cos(angle)).astype(
        jnp.float32
    )


@functools.partial(
    pl.kernel,
    out_type=jax.ShapeDtypeStruct((_SEQ, _DT, _NW, 8, 128), jnp.float32),
    mesh=plsc.VectorSubcoreMesh(core_axis_name="c", subcore_axis_name="s"),
    scratch_types=[
        pltpu.VMEM((_SEQ, _D), jnp.float32),
        pltpu.VMEM((_SEQ, _BPW), jnp.int32),
        pltpu.VMEM((_SC2 * _BPW, _D), jnp.float32),
        pltpu.VMEM((_SC2 * _BPW, _D), jnp.float32),
        pltpu.VMEM((_SC2 * _DT, 8, 129), jnp.float32),
        pltpu.VMEM((_SC2 * _DT, 8, 129), jnp.float32),
        pltpu.SemaphoreType.DMA,
        pltpu.SemaphoreType.DMA,
        pltpu.SemaphoreType.DMA,
        pltpu.SemaphoreType.DMA,
    ],
    compiler_params=pltpu.CompilerParams(
        use_tc_tiling_on_sc=False, needs_layout_passes=False
    ),
)
def _emb_lookup(idx_hbm, table_hbm, pos_hbm, out_hbm, pos_v, idx_v, gbuf0,
                gbuf1, obuf0, obuf1, gsem0, gsem1, osem0, osem1):
    wid = lax.axis_index("s") * _NC + lax.axis_index("c")
    bbase = pl.multiple_of(wid * _BPW, _BPW)
    pltpu.sync_copy(pos_hbm, pos_v)
    pltpu.sync_copy(idx_hbm.at[:, pl.ds(bbase, _BPW)], idx_v)

    def fire_gather(ci, gbuf, sem):
        for t in range(_SC2):
            pltpu.async_copy(
                table_hbm.at[idx_v.at[ci * _SC2 + t]],
                gbuf.at[pl.ds(t * _BPW, _BPW)],
                sem,
            )

    def wait_gather(gbuf, sem):
        for t in range(_SC2):
            pltpu.make_async_copy(
                table_hbm.at[idx_v.at[0]],
                gbuf.at[pl.ds(t * _BPW, _BPW)],
                sem,
            ).wait()

    def fire_out(ci, obuf, sem):
        for t in range(_SC2):
            s = ci * _SC2 + t
            pltpu.async_copy(
                obuf.at[pl.ds(t * _DT, _DT), :, pl.ds(0, 128)],
                out_hbm.at[s, :, wid],
                sem,
            )

    def wait_out(obuf, sem):
        for t in range(_SC2):
            pltpu.make_async_copy(
                obuf.at[pl.ds(t * _DT, _DT), :, pl.ds(0, 128)],
                out_hbm.at[0, :, wid],
                sem,
            ).wait()

    def add_transpose(ci, gbuf, obuf):
        for t in range(_SC2):
            s = ci * _SC2 + t
            pv = [pos_v[s, pl.ds(c * _LANES, _LANES)] for c in range(_VPR)]

            @plsc.parallel_loop(0, _BPW, unroll=8)
            def _(j):
                iota = lax.iota(jnp.int32, _LANES)
                jv = jnp.full((_LANES,), j, jnp.int32)
                d8v = iota % 8
                for c in range(_VPR):
                    rv = iota // 8 + (t * _DT + 2 * c)
                    v = gbuf[t * _BPW + j, pl.ds(c * _LANES, _LANES)] + pv[c]
                    plsc.store_scatter(obuf, [rv, d8v, jv], v)

    fire_gather(0, gbuf0, gsem0)

    def pair_body(k, acc):
        e = k * 2

        fire_gather(e + 1, gbuf1, gsem1)
        wait_gather(gbuf0, gsem0)

        @pl.when(k > 0)
        def _():
            wait_out(obuf0, osem0)

        add_transpose(e, gbuf0, obuf0)
        fire_out(e, obuf0, osem0)

        @pl.when(k < _PAIRS - 1)
        def _():
            fire_gather(e + 2, gbuf0, gsem0)

        wait_gather(gbuf1, gsem1)

        @pl.when(k > 0)
        def _():
            wait_out(obuf1, osem1)

        add_transpose(e + 1, gbuf1, obuf1)
        fire_out(e + 1, obuf1, osem1)
        return acc

    lax.fori_loop(0, _PAIRS, pair_body, 0)
    wait_out(obuf0, osem0)
    wait_out(obuf1, osem1)


def kernel(x, table):
    idx = x.T.astype(jnp.int32)
    pos = _positional(_SEQ, _D, _C)
    out5 = _emb_lookup(idx, table, pos)
    out = out5.transpose(2, 4, 0, 1, 3).reshape(_BATCH, _SEQ, _D)
    return out


# speedup vs baseline: 1.0441x; 1.0441x over previous
import functools

import jax
import jax.numpy as jnp
from jax import lax
from jax.experimental import pallas as pl
from jax.experimental.pallas import tpu as pltpu
from jax.experimental.pallas import tpu_sc as plsc

_VOCAB = 100000
_SEQ = 200
_D = 64
_C = 10000
_BATCH = 4096

_NC = 2
_NS = 16
_NW = _NC * _NS

_BPW = _BATCH // _NW
_DT = _D // 8
_SC2 = 2
_NCHUNK = _SEQ // _SC2
_PAIRS = _NCHUNK // 2
_LANES = 16
_VPR = _D // _LANES


def _positional(seq_len, d_model, c):
    pos = jnp.arange(1, seq_len + 1, dtype=jnp.float32)[:, None]
    j = jnp.arange(d_model)[None, :]
    k = (j + 1) // 2
    angle = pos / jnp.power(jnp.float32(c), k.astype(jnp.float32) / d_model)
    return jnp.where((j % 2) == 0, jnp.sin(angle), jnp.cos(angle)).astype(
        jnp.float32
    )


@functools.partial(
    pl.kernel,
    out_type=jax.ShapeDtypeStruct((_SEQ, _DT, _NW, 8, 128), jnp.float32),
    mesh=plsc.VectorSubcoreMesh(core_axis_name="c", subcore_axis_name="s"),
    scratch_types=[
        pltpu.VMEM((_SEQ, _D), jnp.float32),
        pltpu.VMEM((_SEQ, _BPW), jnp.int32),
        pltpu.VMEM((_SC2 * _BPW, _D), jnp.float32),
        pltpu.VMEM((_SC2 * _BPW, _D), jnp.float32),
        pltpu.VMEM((_SC2 * _DT, 8, 129), jnp.float32),
        pltpu.VMEM((_SC2 * _DT, 8, 129), jnp.float32),
        pltpu.SemaphoreType.DMA,
        pltpu.SemaphoreType.DMA,
        pltpu.SemaphoreType.DMA,
        pltpu.SemaphoreType.DMA,
        pltpu.SemaphoreType.DMA,
    ],
    compiler_params=pltpu.CompilerParams(
        use_tc_tiling_on_sc=False, needs_layout_passes=False
    ),
)
def _emb_lookup(idx_hbm, table_hbm, pos_hbm, out_hbm, pos_v, idx_v, gbuf0,
                gbuf1, obuf0, obuf1, gsem0, gsem1, osem0, osem1, isem):
    wid = lax.axis_index("s") * _NC + lax.axis_index("c")
    bbase = pl.multiple_of(wid * _BPW, _BPW)
    pltpu.sync_copy(pos_hbm, pos_v)
    pltpu.sync_copy(
        idx_hbm.at[pl.ds(0, 8), pl.ds(bbase, _BPW)], idx_v.at[pl.ds(0, 8)]
    )
    pltpu.async_copy(
        idx_hbm.at[pl.ds(8, _SEQ - 8), pl.ds(bbase, _BPW)],
        idx_v.at[pl.ds(8, _SEQ - 8)],
        isem,
    )

    def fire_gather(ci, gbuf, sem):
        for t in range(_SC2):
            pltpu.async_copy(
                table_hbm.at[idx_v.at[ci * _SC2 + t]],
                gbuf.at[pl.ds(t * _BPW, _BPW)],
                sem,
            )

    def wait_gather(gbuf, sem):
        for t in range(_SC2):
            pltpu.make_async_copy(
                table_hbm.at[idx_v.at[0]],
                gbuf.at[pl.ds(t * _BPW, _BPW)],
                sem,
            ).wait()

    def fire_out(ci, obuf, sem):
        for t in range(_SC2):
            s = ci * _SC2 + t
            pltpu.async_copy(
                obuf.at[pl.ds(t * _DT, _DT), :, pl.ds(0, 128)],
                out_hbm.at[s, :, wid],
                sem,
            )

    def wait_out(obuf, sem):
        for t in range(_SC2):
            pltpu.make_async_copy(
                obuf.at[pl.ds(t * _DT, _DT), :, pl.ds(0, 128)],
                out_hbm.at[0, :, wid],
                sem,
            ).wait()

    def add_transpose(ci, gbuf, obuf):
        for t in range(_SC2):
            s = ci * _SC2 + t
            pv = [pos_v[s, pl.ds(c * _LANES, _LANES)] for c in range(_VPR)]

            @plsc.parallel_loop(0, _BPW, unroll=4)
            def _(j):
                iota = lax.iota(jnp.int32, _LANES)
                jv = jnp.full((_LANES,), j, jnp.int32)
                d8v = iota % 8
                for c in range(_VPR):
                    rv = iota // 8 + (t * _DT + 2 * c)
                    v = gbuf[t * _BPW + j, pl.ds(c * _LANES, _LANES)] + pv[c]
                    plsc.store_scatter(obuf, [rv, d8v, jv], v)

    fire_gather(0, gbuf0, gsem0)

    def pair_body(k, acc):
        e = k * 2

        @pl.when(k == 1)
        def _():
            pltpu.make_async_copy(
                idx_hbm.at[pl.ds(8, _SEQ - 8), pl.ds(bbase, _BPW)],
                idx_v.at[pl.ds(8, _SEQ - 8)],
                isem,
            ).wait()

        fire_gather(e + 1, gbuf1, gsem1)
        wait_gather(gbuf0, gsem0)

        @pl.when(k > 0)
        def _():
            wait_out(obuf0, osem0)

        add_transpose(e, gbuf0, obuf0)
        fire_out(e, obuf0, osem0)

        @pl.when(k < _PAIRS - 1)
        def _():
            fire_gather(e + 2, gbuf0, gsem0)

        wait_gather(gbuf1, gsem1)

        @pl.when(k > 0)
        def _():
            wait_out(obuf1, osem1)

        add_transpose(e + 1, gbuf1, obuf1)
        fire_out(e + 1, obuf1, osem1)
        return acc

    lax.fori_loop(0, _PAIRS, pair_body, 0)
    wait_out(obuf0, osem0)
    wait_out(obuf1, osem1)


def kernel(x, table):
    idx = x.T.astype(jnp.int32)
    pos = _positional(_SEQ, _D, _C)
    out5 = _emb_lookup(idx, table, pos)
    out = out5.transpose(2, 4, 0, 1, 3).reshape(_BATCH, _SEQ, _D)
    return out
